# baseline (speedup 1.0000x reference)
import jax
import jax.numpy as jnp
from jax import lax
from jax.experimental import pallas as pl


def _tc_chamfer_body(a_ref, bt_ref, d1_ref, d2_ref):
    i = pl.program_id(1)
    a = a_ref[0]
    bt = bt_ref[0]
    asq = jnp.sum(a * a, axis=1)
    bsq = jnp.sum(bt * bt, axis=0)
    cross = lax.dot_general(a, bt, (((1,), (0,)), ((), ())))
    d = asq[:, None] + bsq[None, :] - 2.0 * cross
    d = jnp.maximum(d, 0.0)
    d1_ref[0, 0, :] = jnp.min(d, axis=1)
    colpart = jnp.min(d, axis=0)

    @pl.when(i == 0)
    def _():
        d2_ref[0, 0, :] = colpart

    @pl.when(i > 0)
    def _():
        d2_ref[0, 0, :] = jnp.minimum(d2_ref[0, 0, :], colpart)


def kernel(xyz1, xyz2):
    B, N, _ = xyz1.shape
    M = xyz2.shape[1]
    IB = 256
    ni = N // IB
    bt = jnp.transpose(xyz2, (0, 2, 1))
    d1, d2 = pl.pallas_call(
        _tc_chamfer_body,
        grid=(B, ni),
        in_specs=[
            pl.BlockSpec((1, IB, 3), lambda b, i: (b, i, 0)),
            pl.BlockSpec((1, 3, M), lambda b, i: (b, 0, 0)),
        ],
        out_specs=[
            pl.BlockSpec((1, 1, IB), lambda b, i: (b * ni + i, 0, 0)),
            pl.BlockSpec((1, 1, M), lambda b, i: (b, 0, 0)),
        ],
        out_shape=[
            jax.ShapeDtypeStruct((B * ni, 1, IB), jnp.float32),
            jax.ShapeDtypeStruct((B, 1, M), jnp.float32),
        ],
    )(xyz1, bt)
    return d1.reshape(B, N), d2.reshape(B, M)


# speedup vs baseline: 1.1894x; 1.1894x over previous
import jax
import jax.numpy as jnp
from jax import lax
from jax.experimental import pallas as pl


def _tc_chamfer_body(a_ref, bt_ref, d1_ref, d2_ref):
    i = pl.program_id(1)
    a = a_ref[0]
    bt = bt_ref[0]
    asq = 0.25 * jnp.sum(a * a, axis=1)
    bsq = jnp.sum(bt * bt, axis=0)
    cross = lax.dot_general(a, bt, (((1,), (0,)), ((), ())))
    d = (cross + asq[:, None]) + bsq[None, :]
    d1_ref[0, 0, :] = jnp.maximum(jnp.min(d, axis=1), 0.0)
    colpart = jnp.maximum(jnp.min(d, axis=0), 0.0)

    @pl.when(i == 0)
    def _():
        d2_ref[0, 0, :] = colpart

    @pl.when(i > 0)
    def _():
        d2_ref[0, 0, :] = jnp.minimum(d2_ref[0, 0, :], colpart)


def kernel(xyz1, xyz2):
    B, N, _ = xyz1.shape
    M = xyz2.shape[1]
    IB = 512
    ni = N // IB
    a2 = -2.0 * xyz1
    bt = jnp.transpose(xyz2, (0, 2, 1))
    d1, d2 = pl.pallas_call(
        _tc_chamfer_body,
        grid=(B, ni),
        in_specs=[
            pl.BlockSpec((1, IB, 3), lambda b, i: (b, i, 0)),
            pl.BlockSpec((1, 3, M), lambda b, i: (b, 0, 0)),
        ],
        out_specs=[
            pl.BlockSpec((1, 1, IB), lambda b, i: (b * ni + i, 0, 0)),
            pl.BlockSpec((1, 1, M), lambda b, i: (b, 0, 0)),
        ],
        out_shape=[
            jax.ShapeDtypeStruct((B * ni, 1, IB), jnp.float32),
            jax.ShapeDtypeStruct((B, 1, M), jnp.float32),
        ],
    )(a2, bt)
    return d1.reshape(B, N), d2.reshape(B, M)
